# Initial kernel scaffold; baseline (speedup 1.0000x reference)
import jax
import jax.numpy as jnp
from jax.experimental import pallas as pl


def kernel(x, params):
    raise NotImplementedError("write your pallas kernel here")


# speedup vs baseline: 5.3321x; 5.3321x over previous
import functools

import jax
import jax.numpy as jnp
from jax.experimental import pallas as pl
from jax.experimental.pallas import tpu as pltpu

_CONFIGS = [(3, 32, 8, 1, -1, 4), (32, 64, 8, 2, -1, 2), (64, 96, 8, 4, -1, 2),
            (96, 128, 12, 4, 120, 2), (128, 160, 12, 6, 120, 2)]
_NUM_CLASSES = 40


def _elu(v):
    return jnp.where(v > 0, v, jnp.exp(jnp.minimum(v, 0.0)) - 1.0)


def _layer_body(*refs, K, D, N, TILE, subset, last):
    it = iter(refs)
    pts_ref = next(it)
    ptsT_ref = next(it)
    fts_ref = next(it)
    sel_oh_ref = next(it) if subset else None
    dense_w = next(it)[...]
    dense_b = next(it)[...]
    d1_w = next(it)[...]
    d1_b = next(it)[...]
    d2_w = next(it)[...]
    d2_b = next(it)[...]
    xt_W2 = next(it)[...]
    xt_b = next(it)[...]
    xt1_w = next(it)[...]
    xt1_b = next(it)[...]
    xt2_w = next(it)[...]
    xt2_b = next(it)[...]
    Wc = next(it)[...]
    bc = next(it)[...]
    if last:
        fc1_w = next(it)[...]
        fc1_b = next(it)[...]
        fc2_w = next(it)[...]
        fc2_b = next(it)[...]
        fc3_w = next(it)[...]
        fc3_b = next(it)[...]
    out_ref = next(it)
    rep_out_ref = next(it) if subset else None
    d2_ref = next(it)
    g_ref = next(it)

    f32 = jnp.float32
    t = pl.program_id(1)
    pts = pts_ref[0]
    ptsT = ptsT_ref[0]
    fts_in = fts_ref[0]

    ftsd = jax.nn.relu(jnp.dot(fts_in, dense_w, preferred_element_type=f32)
                       + dense_b)
    C_half = ftsd.shape[1]

    if subset:
        sel_oh = sel_oh_ref[...]
        rep = jnp.concatenate(
            [jnp.sum(sel_oh * ptsT[c:c + 1, :], axis=1, keepdims=True)
             for c in range(3)], axis=1)
    else:
        rep = pts_ref[0, pl.ds(t * TILE, TILE), :]

    rep2 = jnp.sum(rep * rep, axis=1, keepdims=True)
    pts2 = jnp.sum(ptsT * ptsT, axis=0, keepdims=True)
    dot = jnp.dot(rep.astype(jnp.bfloat16), ptsT.astype(jnp.bfloat16),
                  preferred_element_type=f32)
    d2_ref[...] = rep2 + pts2 - 2.0 * dot

    iota = jax.lax.broadcasted_iota(jnp.int32, (TILE, N), 1)
    cat_src = jnp.concatenate([pts, ftsd], axis=1).astype(jnp.bfloat16)

    def extract(d2v):
        m = jnp.min(d2v, axis=1, keepdims=True)
        c = jnp.min(jnp.where(d2v == m, iota, N), axis=1, keepdims=True)
        oh = iota == c
        return jnp.where(oh, jnp.float32(jnp.inf), d2v), oh

    d2_ref[...] = extract(d2_ref[...])[0]

    def ext_body(j, carry):
        d2v = d2_ref[...]
        d2v, oh = extract(d2v)
        g_ref[pl.ds(j * TILE, TILE), :] = jnp.dot(
            oh.astype(jnp.bfloat16), cat_src, preferred_element_type=f32)
        for _ in range(D - 1):
            d2v, _oh = extract(d2v)
        d2_ref[...] = d2v
        return carry

    jax.lax.fori_loop(0, K, ext_body, 0)

    gathered = [g_ref[k * TILE:(k + 1) * TILE, :] for k in range(K)]
    ptsl = [g[:, 0:3] - rep for g in gathered]
    gfts = [g[:, 3:] for g in gathered]

    ptsl_rows = jnp.concatenate(ptsl, axis=0)
    f_all = jax.nn.relu(jnp.dot(ptsl_rows, d1_w, preferred_element_type=f32)
                        + d1_b)
    f_all = jax.nn.relu(jnp.dot(f_all, d2_w, preferred_element_type=f32)
                        + d2_b)

    cat_k = [jnp.concatenate([f_all[k * TILE:(k + 1) * TILE], gfts[k]], axis=1)
             for k in range(K)]

    ptsl_cols = jnp.concatenate(ptsl, axis=1)
    X = jnp.dot(ptsl_cols, xt_W2, preferred_element_type=f32) + xt_b
    X = _elu(X)
    X = jax.nn.relu(jnp.dot(X, xt1_w, preferred_element_type=f32) + xt1_b)
    X = jnp.dot(X, xt2_w, preferred_element_type=f32) + xt2_b

    ftsX_cols = []
    for k in range(K):
        acc = X[:, k * K:k * K + 1] * cat_k[0]
        for l in range(1, K):
            acc = acc + X[:, k * K + l:k * K + l + 1] * cat_k[l]
        ftsX_cols.append(acc)
    ftsX = jnp.concatenate(ftsX_cols, axis=1)

    out = _elu(jnp.dot(ftsX, Wc, preferred_element_type=f32) + bc)

    if last:
        h = jax.nn.relu(jnp.dot(out, fc1_w, preferred_element_type=f32) + fc1_b)
        h = jax.nn.relu(jnp.dot(h, fc2_w, preferred_element_type=f32) + fc2_b)
        lg = jnp.dot(h, fc3_w, preferred_element_type=f32) + fc3_b
        out_ref[0] = jnp.mean(lg, axis=0, keepdims=True)
    else:
        out_ref[0] = out
        if subset:
            rep_out_ref[0] = rep


def _row(v):
    return v.reshape(1, -1)


def _layer_call(pts, ptsT, fts, lp, cfg, sel_oh, fc=None):
    C_in, C_out, K, D, P_cfg, dm = cfg
    B, N = pts.shape[0], pts.shape[1]
    subset = sel_oh is not None
    P = sel_oh.shape[0] if subset else N
    last = fc is not None
    C_mid = C_out // 4
    C_half = C_out // 2
    C = C_mid + C_half
    TILE = 256 if (not subset and N >= 1024) else P
    T = P // TILE

    xt_W2 = jnp.transpose(lp['xt_w'], (2, 1, 0)).reshape(3 * K, K * K)
    pwr = lp['pw']['w'].reshape(C, dm, C_out)
    Wc = jnp.einsum('cmk,cmo->kco', lp['dw_w'], pwr).reshape(K * C, C_out)
    bc = lp['dw_b'] @ lp['pw']['w'] + lp['pw']['b']

    ops = [pts, ptsT, fts]
    specs = [
        pl.BlockSpec((1, N, 3), lambda b, t: (b, 0, 0)),
        pl.BlockSpec((1, 3, N), lambda b, t: (b, 0, 0)),
        pl.BlockSpec((1, N, C_in), lambda b, t: (b, 0, 0)),
    ]
    if subset:
        ops.append(sel_oh)
        specs.append(pl.BlockSpec((TILE, N), lambda b, t: (t, 0)))

    wlist = [
        lp['dense']['w'], _row(lp['dense']['b']),
        lp['d1']['w'], _row(lp['d1']['b']),
        lp['d2']['w'], _row(lp['d2']['b']),
        xt_W2, _row(lp['xt_b']),
        lp['xt_d1']['w'], _row(lp['xt_d1']['b']),
        lp['xt_d2']['w'], _row(lp['xt_d2']['b']),
        Wc, _row(bc),
    ]
    if last:
        wlist += [fc['fc1']['w'], _row(fc['fc1']['b']),
                  fc['fc2']['w'], _row(fc['fc2']['b']),
                  fc['fc3']['w'], _row(fc['fc3']['b'])]
    for w in wlist:
        ops.append(w)
        specs.append(pl.BlockSpec(w.shape, lambda b, t, nd=w.ndim: (0,) * nd))

    if last:
        out_shape = [jax.ShapeDtypeStruct((B, 1, _NUM_CLASSES), jnp.float32)]
        out_specs = [pl.BlockSpec((1, 1, _NUM_CLASSES), lambda b, t: (b, 0, 0))]
    else:
        out_shape = [jax.ShapeDtypeStruct((B, P, C_out), jnp.float32)]
        out_specs = [pl.BlockSpec((1, TILE, C_out), lambda b, t: (b, t, 0))]
        if subset:
            out_shape.append(jax.ShapeDtypeStruct((B, P, 3), jnp.float32))
            out_specs.append(pl.BlockSpec((1, TILE, 3), lambda b, t: (b, t, 0)))

    body = functools.partial(_layer_body, K=K, D=D, N=N, TILE=TILE,
                             subset=subset, last=last)
    outs = pl.pallas_call(
        body,
        grid=(B, T),
        in_specs=specs,
        out_specs=out_specs,
        out_shape=out_shape,
        scratch_shapes=[
            pltpu.VMEM((TILE, N), jnp.float32),
            pltpu.VMEM((K * TILE, 3 + C_half), jnp.float32),
        ],
    )(*ops)
    return outs


def kernel(x, params):
    pts = jnp.transpose(x, (0, 2, 1))
    fts = pts
    base = jax.random.key(42)
    for i, cfg in enumerate(_CONFIGS):
        C_in, C_out, K, D, P_cfg, dm = cfg
        N = pts.shape[1]
        sel_oh = None
        if 0 < P_cfg < N:
            skey = jax.random.fold_in(base, i)
            sel = jax.random.permutation(skey, N)[:P_cfg]
            sel_oh = (sel[:, None] == jnp.arange(N)[None, :]).astype(jnp.float32)
        ptsT = jnp.transpose(pts, (0, 2, 1))
        fc = ({'fc1': params['fc1'], 'fc2': params['fc2'], 'fc3': params['fc3']}
              if i == len(_CONFIGS) - 1 else None)
        outs = _layer_call(pts, ptsT, fts, params['layers'][i], cfg, sel_oh, fc)
        if fc is not None:
            return outs[0][:, 0, :]
        if sel_oh is not None:
            fts, pts = outs[0], outs[1]
        else:
            fts = outs[0]
    return None


# speedup vs baseline: 6.2208x; 1.1667x over previous
import functools

import jax
import jax.numpy as jnp
from jax.experimental import pallas as pl
from jax.experimental.pallas import tpu as pltpu

_CONFIGS = [(3, 32, 8, 1, -1, 4), (32, 64, 8, 2, -1, 2), (64, 96, 8, 4, -1, 2),
            (96, 128, 12, 4, 120, 2), (128, 160, 12, 6, 120, 2)]
_NUM_CLASSES = 40


def _elu(v):
    return jnp.where(v > 0, v, jnp.exp(jnp.minimum(v, 0.0)) - 1.0)


def _layer_body(*refs, K, D, N, TILE, subset, last):
    it = iter(refs)
    pts_ref = next(it)
    ptsT_ref = next(it)
    fts_ref = next(it)
    sel_oh_ref = next(it) if subset else None
    dense_w = next(it)[...]
    dense_b = next(it)[...]
    d1_w = next(it)[...]
    d1_b = next(it)[...]
    d2_w = next(it)[...]
    d2_b = next(it)[...]
    xt_W2 = next(it)[...]
    xt_b = next(it)[...]
    xt1_w = next(it)[...]
    xt1_b = next(it)[...]
    xt2_w = next(it)[...]
    xt2_b = next(it)[...]
    Wc = next(it)[...]
    bc = next(it)[...]
    if last:
        fc1_w = next(it)[...]
        fc1_b = next(it)[...]
        fc2_w = next(it)[...]
        fc2_b = next(it)[...]
        fc3_w = next(it)[...]
        fc3_b = next(it)[...]
    out_ref = next(it)
    rep_out_ref = next(it) if subset else None
    d2_ref = next(it)
    g_ref = next(it)

    f32 = jnp.float32
    t = pl.program_id(1)
    pts = pts_ref[0]
    ptsT = ptsT_ref[0]
    fts_in = fts_ref[0]

    ftsd = jax.nn.relu(jnp.dot(fts_in, dense_w, preferred_element_type=f32)
                       + dense_b)
    C_half = ftsd.shape[1]

    if subset:
        sel_oh = sel_oh_ref[...]
        rep = jnp.concatenate(
            [jnp.sum(sel_oh * ptsT[c:c + 1, :], axis=1, keepdims=True)
             for c in range(3)], axis=1)
    else:
        rep = pts_ref[0, pl.ds(t * TILE, TILE), :]

    rep2 = jnp.sum(rep * rep, axis=1, keepdims=True)
    pts2 = jnp.sum(ptsT * ptsT, axis=0, keepdims=True)
    dot = jnp.dot(rep.astype(jnp.bfloat16), ptsT.astype(jnp.bfloat16),
                  preferred_element_type=f32)
    d2_ref[...] = rep2 + pts2 - 2.0 * dot

    iota = jax.lax.broadcasted_iota(jnp.int32, (TILE, N), 1)
    cat_src = jnp.concatenate([pts, ftsd], axis=1).astype(jnp.bfloat16)

    def extract(d2v):
        m = jnp.min(d2v, axis=1, keepdims=True)
        c = jnp.min(jnp.where(d2v == m, iota, N), axis=1, keepdims=True)
        oh = iota == c
        return jnp.where(oh, jnp.float32(jnp.inf), d2v), oh

    d2_ref[...] = extract(d2_ref[...])[0]

    def ext_body(j, carry):
        d2v = d2_ref[...]
        d2v, oh = extract(d2v)
        g_ref[pl.ds(j * TILE, TILE), :] = jnp.dot(
            oh.astype(jnp.bfloat16), cat_src, preferred_element_type=f32)
        for _ in range(D - 1):
            d2v, _oh = extract(d2v)
        d2_ref[...] = d2v
        return carry

    jax.lax.fori_loop(0, K, ext_body, 0)

    gathered = [g_ref[k * TILE:(k + 1) * TILE, :] for k in range(K)]
    ptsl = [g[:, 0:3] - rep for g in gathered]
    gfts = [g[:, 3:] for g in gathered]

    ptsl_rows = jnp.concatenate(ptsl, axis=0)
    f_all = jax.nn.relu(jnp.dot(ptsl_rows, d1_w, preferred_element_type=f32)
                        + d1_b)
    f_all = jax.nn.relu(jnp.dot(f_all, d2_w, preferred_element_type=f32)
                        + d2_b)

    cat_k = [jnp.concatenate([f_all[k * TILE:(k + 1) * TILE], gfts[k]], axis=1)
             for k in range(K)]

    ptsl_cols = jnp.concatenate(ptsl, axis=1)
    X = jnp.dot(ptsl_cols, xt_W2, preferred_element_type=f32) + xt_b
    X = _elu(X)
    X = jax.nn.relu(jnp.dot(X, xt1_w, preferred_element_type=f32) + xt1_b)
    X = jnp.dot(X, xt2_w, preferred_element_type=f32) + xt2_b

    ftsX_cols = []
    for k in range(K):
        acc = X[:, k * K:k * K + 1] * cat_k[0]
        for l in range(1, K):
            acc = acc + X[:, k * K + l:k * K + l + 1] * cat_k[l]
        ftsX_cols.append(acc)
    ftsX = jnp.concatenate(ftsX_cols, axis=1)

    out = _elu(jnp.dot(ftsX, Wc, preferred_element_type=f32) + bc)

    if last:
        h = jax.nn.relu(jnp.dot(out, fc1_w, preferred_element_type=f32) + fc1_b)
        h = jax.nn.relu(jnp.dot(h, fc2_w, preferred_element_type=f32) + fc2_b)
        lg = jnp.dot(h, fc3_w, preferred_element_type=f32) + fc3_b
        out_ref[0] = jnp.mean(lg, axis=0, keepdims=True)
    else:
        out_ref[0] = out
        if subset:
            rep_out_ref[0] = rep


def _row(v):
    return v.reshape(1, -1)


def _layer_call(pts, ptsT, fts, lp, cfg, sel_oh, fc=None):
    C_in, C_out, K, D, P_cfg, dm = cfg
    B, N = pts.shape[0], pts.shape[1]
    subset = sel_oh is not None
    P = sel_oh.shape[0] if subset else N
    last = fc is not None
    C_mid = C_out // 4
    C_half = C_out // 2
    C = C_mid + C_half
    TILE = 512 if (not subset and N >= 1024) else P
    T = P // TILE

    xt_W2 = jnp.transpose(lp['xt_w'], (2, 1, 0)).reshape(3 * K, K * K)
    pwr = lp['pw']['w'].reshape(C, dm, C_out)
    Wc = jnp.einsum('cmk,cmo->kco', lp['dw_w'], pwr).reshape(K * C, C_out)
    bc = lp['dw_b'] @ lp['pw']['w'] + lp['pw']['b']

    ops = [pts, ptsT, fts]
    specs = [
        pl.BlockSpec((1, N, 3), lambda b, t: (b, 0, 0)),
        pl.BlockSpec((1, 3, N), lambda b, t: (b, 0, 0)),
        pl.BlockSpec((1, N, C_in), lambda b, t: (b, 0, 0)),
    ]
    if subset:
        ops.append(sel_oh)
        specs.append(pl.BlockSpec((TILE, N), lambda b, t: (t, 0)))

    wlist = [
        lp['dense']['w'], _row(lp['dense']['b']),
        lp['d1']['w'], _row(lp['d1']['b']),
        lp['d2']['w'], _row(lp['d2']['b']),
        xt_W2, _row(lp['xt_b']),
        lp['xt_d1']['w'], _row(lp['xt_d1']['b']),
        lp['xt_d2']['w'], _row(lp['xt_d2']['b']),
        Wc, _row(bc),
    ]
    if last:
        wlist += [fc['fc1']['w'], _row(fc['fc1']['b']),
                  fc['fc2']['w'], _row(fc['fc2']['b']),
                  fc['fc3']['w'], _row(fc['fc3']['b'])]
    for w in wlist:
        ops.append(w)
        specs.append(pl.BlockSpec(w.shape, lambda b, t, nd=w.ndim: (0,) * nd))

    if last:
        out_shape = [jax.ShapeDtypeStruct((B, 1, _NUM_CLASSES), jnp.float32)]
        out_specs = [pl.BlockSpec((1, 1, _NUM_CLASSES), lambda b, t: (b, 0, 0))]
    else:
        out_shape = [jax.ShapeDtypeStruct((B, P, C_out), jnp.float32)]
        out_specs = [pl.BlockSpec((1, TILE, C_out), lambda b, t: (b, t, 0))]
        if subset:
            out_shape.append(jax.ShapeDtypeStruct((B, P, 3), jnp.float32))
            out_specs.append(pl.BlockSpec((1, TILE, 3), lambda b, t: (b, t, 0)))

    body = functools.partial(_layer_body, K=K, D=D, N=N, TILE=TILE,
                             subset=subset, last=last)
    outs = pl.pallas_call(
        body,
        grid=(B, T),
        in_specs=specs,
        out_specs=out_specs,
        out_shape=out_shape,
        scratch_shapes=[
            pltpu.VMEM((TILE, N), jnp.float32),
            pltpu.VMEM((K * TILE, 3 + C_half), jnp.float32),
        ],
    )(*ops)
    return outs


def kernel(x, params):
    pts = jnp.transpose(x, (0, 2, 1))
    fts = pts
    base = jax.random.key(42)
    for i, cfg in enumerate(_CONFIGS):
        C_in, C_out, K, D, P_cfg, dm = cfg
        N = pts.shape[1]
        sel_oh = None
        if 0 < P_cfg < N:
            skey = jax.random.fold_in(base, i)
            sel = jax.random.permutation(skey, N)[:P_cfg]
            sel_oh = (sel[:, None] == jnp.arange(N)[None, :]).astype(jnp.float32)
        ptsT = jnp.transpose(pts, (0, 2, 1))
        fc = ({'fc1': params['fc1'], 'fc2': params['fc2'], 'fc3': params['fc3']}
              if i == len(_CONFIGS) - 1 else None)
        outs = _layer_call(pts, ptsT, fts, params['layers'][i], cfg, sel_oh, fc)
        if fc is not None:
            return outs[0][:, 0, :]
        if sel_oh is not None:
            fts, pts = outs[0], outs[1]
        else:
            fts = outs[0]
    return None


# speedup vs baseline: 6.3997x; 1.0288x over previous
import functools

import jax
import jax.numpy as jnp
from jax.experimental import pallas as pl
from jax.experimental.pallas import tpu as pltpu

_CONFIGS = [(3, 32, 8, 1, -1, 4), (32, 64, 8, 2, -1, 2), (64, 96, 8, 4, -1, 2),
            (96, 128, 12, 4, 120, 2), (128, 160, 12, 6, 120, 2)]
_NUM_CLASSES = 40


def _elu(v):
    return jnp.where(v > 0, v, jnp.exp(jnp.minimum(v, 0.0)) - 1.0)


def _layer_body(*refs, K, D, N, TILE, subset, last):
    it = iter(refs)
    pts_ref = next(it)
    ptsT_ref = next(it)
    fts_ref = next(it)
    sel_oh_ref = next(it) if subset else None
    dense_w = next(it)[...]
    dense_b = next(it)[...]
    d1_w = next(it)[...]
    d1_b = next(it)[...]
    d2_w = next(it)[...]
    d2_b = next(it)[...]
    xt_W2 = next(it)[...]
    xt_b = next(it)[...]
    xt1_w = next(it)[...]
    xt1_b = next(it)[...]
    xt2_w = next(it)[...]
    xt2_b = next(it)[...]
    Wc = next(it)[...]
    bc = next(it)[...]
    if last:
        fc1_w = next(it)[...]
        fc1_b = next(it)[...]
        fc2_w = next(it)[...]
        fc2_b = next(it)[...]
        fc3_w = next(it)[...]
        fc3_b = next(it)[...]
    out_ref = next(it)
    rep_out_ref = next(it) if subset else None
    d2_ref = next(it)
    g_ref = next(it)

    f32 = jnp.float32
    t = pl.program_id(1)
    pts = pts_ref[0]
    ptsT = ptsT_ref[0]
    fts_in = fts_ref[0]

    ftsd = jax.nn.relu(jnp.dot(fts_in, dense_w, preferred_element_type=f32)
                       + dense_b)
    C_half = ftsd.shape[1]

    if subset:
        sel_oh = sel_oh_ref[...]
        rep = jnp.concatenate(
            [jnp.sum(sel_oh * ptsT[c:c + 1, :], axis=1, keepdims=True)
             for c in range(3)], axis=1)
    else:
        rep = pts_ref[0, pl.ds(t * TILE, TILE), :]

    rep2 = jnp.sum(rep * rep, axis=1, keepdims=True)
    pts2 = jnp.sum(ptsT * ptsT, axis=0, keepdims=True)
    dot = jnp.dot(rep.astype(jnp.bfloat16), ptsT.astype(jnp.bfloat16),
                  preferred_element_type=f32)
    d2_ref[...] = rep2 + pts2 - 2.0 * dot

    iota = jax.lax.broadcasted_iota(jnp.int32, (TILE, N), 1)
    cat_src = jnp.concatenate([pts, ftsd], axis=1).astype(jnp.bfloat16)

    def extract(d2v):
        m = jnp.min(d2v, axis=1, keepdims=True)
        c = jnp.min(jnp.where(d2v == m, iota, N), axis=1, keepdims=True)
        oh = iota == c
        return jnp.where(oh, jnp.float32(jnp.inf), d2v), oh

    d2_ref[...] = extract(d2_ref[...])[0]

    def ext_body(j, carry):
        d2v = d2_ref[...]
        d2v, oh = extract(d2v)
        g_ref[pl.ds(j * TILE, TILE), :] = jnp.dot(
            oh.astype(jnp.bfloat16), cat_src, preferred_element_type=f32)
        for _ in range(D - 1):
            d2v, _oh = extract(d2v)
        d2_ref[...] = d2v
        return carry

    jax.lax.fori_loop(0, K, ext_body, 0)

    gathered = [g_ref[k * TILE:(k + 1) * TILE, :] for k in range(K)]
    ptsl = [g[:, 0:3] - rep for g in gathered]
    gfts = [g[:, 3:] for g in gathered]

    ptsl_rows = jnp.concatenate(ptsl, axis=0)
    f_all = jax.nn.relu(jnp.dot(ptsl_rows, d1_w, preferred_element_type=f32)
                        + d1_b)
    f_all = jax.nn.relu(jnp.dot(f_all, d2_w, preferred_element_type=f32)
                        + d2_b)

    cat_k = [jnp.concatenate([f_all[k * TILE:(k + 1) * TILE], gfts[k]], axis=1)
             for k in range(K)]

    ptsl_cols = jnp.concatenate(ptsl, axis=1)
    X = jnp.dot(ptsl_cols, xt_W2, preferred_element_type=f32) + xt_b
    X = _elu(X)
    X = jax.nn.relu(jnp.dot(X, xt1_w, preferred_element_type=f32) + xt1_b)
    X = jnp.dot(X, xt2_w, preferred_element_type=f32) + xt2_b

    ftsX_cols = []
    for k in range(K):
        acc = X[:, k * K:k * K + 1] * cat_k[0]
        for l in range(1, K):
            acc = acc + X[:, k * K + l:k * K + l + 1] * cat_k[l]
        ftsX_cols.append(acc)
    ftsX = jnp.concatenate(ftsX_cols, axis=1)

    out = _elu(jnp.dot(ftsX, Wc, preferred_element_type=f32) + bc)

    if last:
        h = jax.nn.relu(jnp.dot(out, fc1_w, preferred_element_type=f32) + fc1_b)
        h = jax.nn.relu(jnp.dot(h, fc2_w, preferred_element_type=f32) + fc2_b)
        lg = jnp.dot(h, fc3_w, preferred_element_type=f32) + fc3_b
        out_ref[0] = jnp.mean(lg, axis=0, keepdims=True)
    else:
        out_ref[0] = out
        if subset:
            rep_out_ref[0] = rep


def _row(v):
    return v.reshape(1, -1)


def _layer_call(pts, ptsT, fts, lp, cfg, sel_oh, fc=None):
    C_in, C_out, K, D, P_cfg, dm = cfg
    B, N = pts.shape[0], pts.shape[1]
    subset = sel_oh is not None
    P = sel_oh.shape[0] if subset else N
    last = fc is not None
    C_mid = C_out // 4
    C_half = C_out // 2
    C = C_mid + C_half
    TILE = 1024 if (not subset and N >= 1024) else P
    T = P // TILE

    xt_W2 = jnp.transpose(lp['xt_w'], (2, 1, 0)).reshape(3 * K, K * K)
    pwr = lp['pw']['w'].reshape(C, dm, C_out)
    Wc = jnp.einsum('cmk,cmo->kco', lp['dw_w'], pwr).reshape(K * C, C_out)
    bc = lp['dw_b'] @ lp['pw']['w'] + lp['pw']['b']

    ops = [pts, ptsT, fts]
    specs = [
        pl.BlockSpec((1, N, 3), lambda b, t: (b, 0, 0)),
        pl.BlockSpec((1, 3, N), lambda b, t: (b, 0, 0)),
        pl.BlockSpec((1, N, C_in), lambda b, t: (b, 0, 0)),
    ]
    if subset:
        ops.append(sel_oh)
        specs.append(pl.BlockSpec((TILE, N), lambda b, t: (t, 0)))

    wlist = [
        lp['dense']['w'], _row(lp['dense']['b']),
        lp['d1']['w'], _row(lp['d1']['b']),
        lp['d2']['w'], _row(lp['d2']['b']),
        xt_W2, _row(lp['xt_b']),
        lp['xt_d1']['w'], _row(lp['xt_d1']['b']),
        lp['xt_d2']['w'], _row(lp['xt_d2']['b']),
        Wc, _row(bc),
    ]
    if last:
        wlist += [fc['fc1']['w'], _row(fc['fc1']['b']),
                  fc['fc2']['w'], _row(fc['fc2']['b']),
                  fc['fc3']['w'], _row(fc['fc3']['b'])]
    for w in wlist:
        ops.append(w)
        specs.append(pl.BlockSpec(w.shape, lambda b, t, nd=w.ndim: (0,) * nd))

    if last:
        out_shape = [jax.ShapeDtypeStruct((B, 1, _NUM_CLASSES), jnp.float32)]
        out_specs = [pl.BlockSpec((1, 1, _NUM_CLASSES), lambda b, t: (b, 0, 0))]
    else:
        out_shape = [jax.ShapeDtypeStruct((B, P, C_out), jnp.float32)]
        out_specs = [pl.BlockSpec((1, TILE, C_out), lambda b, t: (b, t, 0))]
        if subset:
            out_shape.append(jax.ShapeDtypeStruct((B, P, 3), jnp.float32))
            out_specs.append(pl.BlockSpec((1, TILE, 3), lambda b, t: (b, t, 0)))

    body = functools.partial(_layer_body, K=K, D=D, N=N, TILE=TILE,
                             subset=subset, last=last)
    outs = pl.pallas_call(
        body,
        grid=(B, T),
        in_specs=specs,
        out_specs=out_specs,
        out_shape=out_shape,
        scratch_shapes=[
            pltpu.VMEM((TILE, N), jnp.float32),
            pltpu.VMEM((K * TILE, 3 + C_half), jnp.float32),
        ],
    )(*ops)
    return outs


def kernel(x, params):
    pts = jnp.transpose(x, (0, 2, 1))
    fts = pts
    base = jax.random.key(42)
    for i, cfg in enumerate(_CONFIGS):
        C_in, C_out, K, D, P_cfg, dm = cfg
        N = pts.shape[1]
        sel_oh = None
        if 0 < P_cfg < N:
            skey = jax.random.fold_in(base, i)
            sel = jax.random.permutation(skey, N)[:P_cfg]
            sel_oh = (sel[:, None] == jnp.arange(N)[None, :]).astype(jnp.float32)
        ptsT = jnp.transpose(pts, (0, 2, 1))
        fc = ({'fc1': params['fc1'], 'fc2': params['fc2'], 'fc3': params['fc3']}
              if i == len(_CONFIGS) - 1 else None)
        outs = _layer_call(pts, ptsT, fts, params['layers'][i], cfg, sel_oh, fc)
        if fc is not None:
            return outs[0][:, 0, :]
        if sel_oh is not None:
            fts, pts = outs[0], outs[1]
        else:
            fts = outs[0]
    return None
